# baseline (speedup 1.0000x reference)
import functools

import jax
import jax.numpy as jnp
from jax import lax
from jax.experimental import pallas as pl
from jax.experimental.pallas import tpu as pltpu
from jax.experimental.pallas import tpu_sc as plsc

VOCAB = 1000000
D = 32
T = 200
B = 4096
LANES = 16
NBUF = 3
CH = T


@functools.lru_cache(maxsize=1)
def _build():
  info = plsc.get_sparse_core_info()
  nc, ns = info.num_cores, info.num_subcores
  nw = nc * ns
  rows_per_w = B // nw
  ngroups, rem = divmod(rows_per_w, NBUF)

  mesh = plsc.VectorSubcoreMesh(core_axis_name="c", subcore_axis_name="s")

  @functools.partial(
      pl.kernel,
      mesh=mesh,
      out_type=jax.ShapeDtypeStruct((B * T, D), jnp.float32),
      scratch_types=(
          [pltpu.VMEM((T, D), jnp.float32)]
          + [pltpu.VMEM((CH,), jnp.int32)] * NBUF
          + [pltpu.VMEM((CH, 128), jnp.float32)] * NBUF
          + [pltpu.VMEM((CH, D), jnp.float32)] * NBUF
          + [pltpu.SemaphoreType.DMA] * (2 * NBUF)
      ),
      compiler_params=pltpu.CompilerParams(use_tc_tiling_on_sc=False),
  )
  def emb_kernel(x_hbm, tt_hbm, pt_hbm, out_hbm, pos_v, *rest):
    idx_v = rest[:NBUF]
    tok_v = rest[NBUF:2 * NBUF]
    o_v = rest[2 * NBUF:3 * NBUF]
    gsem = rest[3 * NBUF:4 * NBUF]
    osem = rest[4 * NBUF:]
    wid = lax.axis_index("s") * nc + lax.axis_index("c")
    w_base = wid * (rows_per_w * T)
    pltpu.sync_copy(pt_hbm, pos_v)

    def do_group(cs):
      gd = []
      for k, c in enumerate(cs):
        base = w_base + c * CH
        pltpu.sync_copy(x_hbm.at[pl.ds(base, CH)], idx_v[k])
        gd.append(pltpu.async_copy(tt_hbm.at[idx_v[k]], tok_v[k], gsem[k]))
      od = []
      for k, c in enumerate(cs):
        gd[k].wait()

        def add_fn(i, carry, k=k):
          ii = i * 8
          tb = tok_v[k]
          ob = o_v[k]
          for j in range(8):
            for h in range(2):
              s = pl.ds(h * LANES, LANES)
              ob[ii + j, s] = tb[ii + j, s] + pos_v[ii + j, s]
          return carry

        lax.fori_loop(0, CH // 8, add_fn, 0)
        base = w_base + c * CH
        od.append(pltpu.async_copy(o_v[k], out_hbm.at[pl.ds(base, CH)],
                                   osem[k]))
      for d in od:
        d.wait()

    def group_fn(g, carry):
      do_group([g * NBUF + k for k in range(NBUF)])
      return carry

    lax.fori_loop(0, ngroups, group_fn, 0)
    if rem:
      do_group([ngroups * NBUF + k for k in range(rem)])

  return emb_kernel


def kernel(x, token_table, position_table):
  ttp = jnp.pad(token_table, ((0, 0), (0, 128 - D)))
  out_flat = _build()(x.reshape(B * T).astype(jnp.int32), ttp,
                      position_table)
  return out_flat.reshape(B, T, D)


# speedup vs baseline: 1.2767x; 1.2767x over previous
import functools

import jax
import jax.numpy as jnp
from jax import lax
from jax.experimental import pallas as pl
from jax.experimental.pallas import tpu as pltpu
from jax.experimental.pallas import tpu_sc as plsc

VOCAB = 1000000
D = 32
T = 200
B = 4096
LANES = 16
R = 4
NBUF = 4
PREF = 2
CH = R * T


@functools.lru_cache(maxsize=1)
def _build():
  info = plsc.get_sparse_core_info()
  nc, ns = info.num_cores, info.num_subcores
  nw = nc * ns
  rows_per_w = B // nw
  nch = rows_per_w // R

  mesh = plsc.VectorSubcoreMesh(core_axis_name="c", subcore_axis_name="s")

  @functools.partial(
      pl.kernel,
      mesh=mesh,
      out_type=jax.ShapeDtypeStruct((B * T, D), jnp.float32),
      scratch_types=(
          [pltpu.VMEM((T, D), jnp.float32)]
          + [pltpu.VMEM((CH,), jnp.int32)] * NBUF
          + [pltpu.VMEM((CH, D), jnp.float32)] * NBUF
          + [pltpu.SemaphoreType.DMA] * (2 * NBUF)
      ),
      compiler_params=pltpu.CompilerParams(use_tc_tiling_on_sc=False),
  )
  def emb_kernel(x_hbm, tt_hbm, pt_hbm, out_hbm, pos_v, *rest):
    idx_v = rest[:NBUF]
    tok_v = rest[NBUF:2 * NBUF]
    gsem = rest[2 * NBUF:3 * NBUF]
    osem = rest[3 * NBUF:]
    wid = lax.axis_index("s") * nc + lax.axis_index("c")
    w_base = wid * (rows_per_w * T)
    pltpu.sync_copy(pt_hbm, pos_v)

    gather_d = [None] * NBUF
    out_d = [None] * NBUF

    def start_chunk(c):
      b = c % NBUF
      if out_d[b] is not None:
        out_d[b].wait()
      base = w_base + c * CH
      pltpu.sync_copy(x_hbm.at[pl.ds(base, CH)], idx_v[b])
      gather_d[b] = pltpu.async_copy(tt_hbm.at[idx_v[b]], tok_v[b], gsem[b])

    for p in range(PREF):
      start_chunk(p)

    for c in range(nch):
      if c + PREF < nch:
        start_chunk(c + PREF)
      b = c % NBUF
      gather_d[b].wait()
      tb = tok_v[b]

      @plsc.parallel_loop(0, CH // 8)
      def add_fn(i, tb=tb):
        pp = lax.rem(i, T // 8) * 8
        ii = i * 8
        for j in range(8):
          for h in range(2):
            s = pl.ds(h * LANES, LANES)
            tb[ii + j, s] = tb[ii + j, s] + pos_v[pp + j, s]

      base = w_base + c * CH
      out_d[b] = pltpu.async_copy(tb, out_hbm.at[pl.ds(base, CH)],
                                  osem[b])

    for b in range(NBUF):
      if out_d[b] is not None:
        out_d[b].wait()

  return emb_kernel


def kernel(x, token_table, position_table):
  out_flat = _build()(x.reshape(B * T).astype(jnp.int32), token_table,
                      position_table)
  return out_flat.reshape(B, T, D)
